# Initial kernel scaffold; baseline (speedup 1.0000x reference)
import jax
import jax.numpy as jnp
from jax.experimental import pallas as pl


def kernel(x, w1, b1, w2, b2, wf1, bf1, wf2, bf2):
    raise NotImplementedError("write your pallas kernel here")


# speedup vs baseline: 8.2257x; 8.2257x over previous
import functools

import jax
import jax.numpy as jnp
from jax.experimental import pallas as pl
from jax.experimental.pallas import tpu as pltpu


def _fused_cnn_kernel(bb, ch,
                      xt_ref,
                      a1_ref,
                      b1t_ref,
                      a2_ref,
                      b2t_ref,
                      wf1_ref,
                      bf1_ref,
                      wf2_ref,
                      bf2_ref,
                      out_ref,
                      flat_ref):
    f32 = jnp.float32
    n = ch

    for c in range(bb // ch):
        xc = xt_ref[c * n:(c + 1) * n]
        xp = jnp.concatenate(
            [jnp.zeros((n, 100, 1), f32), xc, jnp.zeros((n, 100, 3), f32)],
            axis=2)
        xp = jnp.concatenate(
            [jnp.zeros((n, 1, 16), f32), xp, jnp.zeros((n, 1, 16), f32)],
            axis=1)
        pat1 = jnp.concatenate(
            [xp[:, 0:100, :], xp[:, 1:101, :], xp[:, 2:102, :]],
            axis=2).reshape(n * 100, 48)
        y1 = jnp.dot(pat1, a1_ref[...], preferred_element_type=f32)
        y1 = jnp.maximum(y1 + b1t_ref[...], 0.0)
        p1 = jnp.maximum(y1[:, 0:96], y1[:, 96:192]).reshape(n, 100, 96)
        p1 = jnp.max(p1.reshape(n, 50, 2, 96), axis=2)

        p1p = jnp.concatenate(
            [jnp.zeros((n, 1, 96), f32), p1, jnp.zeros((n, 1, 96), f32)],
            axis=1)
        z16 = jnp.zeros((n, 50, 16), f32)
        parts = []
        for dx in range(3):
            parts += [z16, p1p[:, dx:dx + 50, :], z16]
        pat2 = jnp.concatenate(parts, axis=2).reshape(n * 50, 384)
        y2 = jnp.dot(pat2, a2_ref[...], preferred_element_type=f32)
        y2 = jnp.maximum(y2 + b2t_ref[...], 0.0)
        p2 = jnp.maximum(y2[:, 0:96], y2[:, 96:192]).reshape(n, 50, 96)
        p2 = jnp.max(p2.reshape(n, 25, 2, 96), axis=2)
        flat_ref[c * n:(c + 1) * n] = p2.reshape(n, 2400)

    h1 = jnp.dot(flat_ref[...], wf1_ref[...], preferred_element_type=f32)
    h1 = jnp.maximum(h1 + bf1_ref[...], 0.0)
    out_ref[...] = jnp.dot(h1, wf2_ref[...],
                           preferred_element_type=f32) + bf2_ref[...]


def _even_odd_cols(a, groups, width):
    k = a.shape[0]
    return a.reshape(k, groups, 2, width).transpose(0, 2, 1, 3).reshape(k, -1)


def _prep_conv1(w1):
    lanes = jnp.arange(16)
    hout = jnp.arange(12)
    dy = lanes[:, None] - hout[None, :]
    mask = (dy >= 0) & (dy <= 2) & (lanes[:, None] <= 13)
    w1r = w1.reshape(3, 3, 16)
    g = w1r[jnp.clip(dy, 0, 2)]
    a1 = jnp.where(mask[:, :, None, None], g, 0.0)
    a1 = a1.transpose(2, 0, 1, 3).reshape(48, 192)
    return _even_odd_cols(a1, 6, 16)


def _prep_conv2(w2):
    hp2 = jnp.arange(8)
    hout = jnp.arange(6)
    dy = hp2[:, None] - hout[None, :]
    mask = (dy >= 0) & (dy <= 2)
    w2r = w2.reshape(3, 3, 16, 32)
    g = w2r[jnp.clip(dy, 0, 2)]
    a2 = jnp.where(mask[:, :, None, None, None], g, 0.0)
    a2 = a2.transpose(2, 0, 3, 1, 4).reshape(384, 192)
    return _even_odd_cols(a2, 3, 32)


def kernel(x, w1, b1, w2, b2, wf1, bf1, wf2, bf2):
    B = x.shape[0]
    C = wf2.shape[1]
    bb = 64
    while B % bb:
        bb //= 2
    ch = min(32, bb)
    xt = jnp.swapaxes(x.reshape(B, 12, 100), 1, 2)

    a1 = _prep_conv1(w1)
    b1t = jnp.tile(b1, (1, 12))
    a2 = _prep_conv2(w2)
    b2t = jnp.tile(b2, (1, 6))
    wf1r = wf1.reshape(3, 25, 32, 128).transpose(1, 0, 2, 3).reshape(2400, 128)

    out = pl.pallas_call(
        functools.partial(_fused_cnn_kernel, bb, ch),
        out_shape=jax.ShapeDtypeStruct((B, C), jnp.float32),
        grid=(B // bb,),
        in_specs=[
            pl.BlockSpec((bb, 100, 12), lambda i: (i, 0, 0)),
            pl.BlockSpec((48, 192), lambda i: (0, 0)),
            pl.BlockSpec((1, 192), lambda i: (0, 0)),
            pl.BlockSpec((384, 192), lambda i: (0, 0)),
            pl.BlockSpec((1, 192), lambda i: (0, 0)),
            pl.BlockSpec((2400, 128), lambda i: (0, 0)),
            pl.BlockSpec((1, 128), lambda i: (0, 0)),
            pl.BlockSpec((128, C), lambda i: (0, 0)),
            pl.BlockSpec((1, C), lambda i: (0, 0)),
        ],
        out_specs=pl.BlockSpec((bb, C), lambda i: (i, 0)),
        scratch_shapes=[pltpu.VMEM((bb, 2400), jnp.float32)],
        compiler_params=pltpu.CompilerParams(
            dimension_semantics=("parallel",)),
    )(xt, a1, b1t, a2, b2t, wf1r, bf1, wf2, bf2)
    return out


# speedup vs baseline: 11.7958x; 1.4340x over previous
import functools

import jax
import jax.numpy as jnp
from jax.experimental import pallas as pl
from jax.experimental.pallas import tpu as pltpu


def _fused_cnn_kernel(bb, ch,
                      xt_ref,
                      a1e_ref,
                      a1o_ref,
                      b1t_ref,
                      a2e_ref,
                      a2o_ref,
                      b2t_ref,
                      wf1_ref,
                      bf1_ref,
                      wf2_ref,
                      bf2_ref,
                      out_ref,
                      pat1_ref,
                      p1h_ref,
                      pat2_ref,
                      p2h_ref,
                      flat_ref):
    f32 = jnp.float32
    bf16 = jnp.bfloat16
    n = ch

    for c in range(bb // ch):
        s = c * n
        xc = xt_ref[s:s + n]
        x16 = jnp.concatenate([xc, jnp.zeros((n, 100, 4), bf16)], axis=2)
        z1 = jnp.zeros((n, 1, 16), bf16)
        pat1_ref[:, 1:100, 0:16] = x16[:, 0:99]
        pat1_ref[:, 0:1, 0:16] = z1
        pat1_ref[:, :, 16:32] = x16
        pat1_ref[:, 0:99, 32:48] = x16[:, 1:100]
        pat1_ref[:, 99:100, 32:48] = z1

        pat1 = pat1_ref[...].reshape(n * 100, 48)
        y1 = jnp.maximum(
            jnp.dot(pat1, a1e_ref[...], preferred_element_type=f32),
            jnp.dot(pat1, a1o_ref[...], preferred_element_type=f32))
        p1h_ref[...] = y1.reshape(n, 100, 128)
        p1 = jnp.maximum(p1h_ref[:, pl.ds(0, 50, stride=2), :],
                         p1h_ref[:, pl.ds(1, 50, stride=2), :])
        p1 = jnp.maximum(p1 + b1t_ref[...], 0.0).astype(bf16)

        zr = jnp.zeros((n, 1, 128), bf16)
        pat2_ref[:, 1:50, 0:128] = p1[:, 0:49]
        pat2_ref[:, 0:1, 0:128] = zr
        pat2_ref[:, :, 128:256] = p1
        pat2_ref[:, 0:49, 256:384] = p1[:, 1:50]
        pat2_ref[:, 49:50, 256:384] = zr

        pat2 = pat2_ref[...].reshape(n * 50, 384)
        y2 = jnp.maximum(
            jnp.dot(pat2, a2e_ref[...], preferred_element_type=f32),
            jnp.dot(pat2, a2o_ref[...], preferred_element_type=f32))
        p2h_ref[...] = y2.reshape(n, 50, 128)
        p2 = jnp.maximum(p2h_ref[:, pl.ds(0, 25, stride=2), :],
                         p2h_ref[:, pl.ds(1, 25, stride=2), :])
        p2 = jnp.maximum(p2 + b2t_ref[...], 0.0).astype(bf16)
        flat_ref[s:s + n] = p2.reshape(n, 3200)

    h1 = jnp.dot(flat_ref[...], wf1_ref[...], preferred_element_type=f32)
    h1 = jnp.maximum(h1 + bf1_ref[...], 0.0).astype(jnp.bfloat16)
    out_ref[...] = jnp.dot(h1, wf2_ref[...],
                           preferred_element_type=f32) + bf2_ref[...]


def _split_parity_cols(a, groups, width):
    k = a.shape[0]
    a = a.reshape(k, groups, 2, width).transpose(0, 2, 1, 3).reshape(k, 2, -1)
    a = jnp.pad(a, ((0, 0), (0, 0), (0, 128 - groups * width)))
    return a[:, 0, :].astype(jnp.bfloat16), a[:, 1, :].astype(jnp.bfloat16)


def _prep_conv1(w1):
    lanes = jnp.arange(16)
    hout = jnp.arange(12)
    dy = lanes[:, None] - hout[None, :] + 1
    mask = (dy >= 0) & (dy <= 2) & (lanes[:, None] < 12)
    w1r = w1.reshape(3, 3, 16)
    g = w1r[jnp.clip(dy, 0, 2)]
    a1 = jnp.where(mask[:, :, None, None], g, 0.0)
    a1 = a1.transpose(2, 0, 1, 3).reshape(48, 192)
    return _split_parity_cols(a1, 6, 16)


def _prep_conv2(w2):
    h2in = jnp.arange(8)
    hout = jnp.arange(6)
    dy = h2in[:, None] - hout[None, :] + 1
    mask = (dy >= 0) & (dy <= 2) & (h2in[:, None] < 6)
    w2r = w2.reshape(3, 3, 16, 32)
    g = w2r[jnp.clip(dy, 0, 2)]
    a2 = jnp.where(mask[:, :, None, None, None], g, 0.0)
    a2 = a2.transpose(2, 0, 3, 1, 4).reshape(384, 192)
    return _split_parity_cols(a2, 3, 32)


def kernel(x, w1, b1, w2, b2, wf1, bf1, wf2, bf2):
    B = x.shape[0]
    C = wf2.shape[1]
    bb = 64
    while B % bb:
        bb //= 2
    ch = min(32, bb)
    bf16 = jnp.bfloat16
    xt = jnp.swapaxes(x.reshape(B, 12, 100), 1, 2).astype(bf16)

    a1e, a1o = _prep_conv1(w1)
    b1t = jnp.pad(jnp.tile(b1, (1, 6)), ((0, 0), (0, 32)))
    a2e, a2o = _prep_conv2(w2)
    b2t = jnp.pad(jnp.tile(b2, (1, 3)), ((0, 0), (0, 32)))
    wf1r = wf1.reshape(3, 25, 32, 128).transpose(1, 0, 2, 3).reshape(25, 96, 128)
    wf1r = jnp.pad(wf1r, ((0, 0), (0, 32), (0, 0))).reshape(3200, 128).astype(bf16)

    out = pl.pallas_call(
        functools.partial(_fused_cnn_kernel, bb, ch),
        out_shape=jax.ShapeDtypeStruct((B, C), jnp.float32),
        grid=(B // bb,),
        in_specs=[
            pl.BlockSpec((bb, 100, 12), lambda i: (i, 0, 0)),
            pl.BlockSpec((48, 128), lambda i: (0, 0)),
            pl.BlockSpec((48, 128), lambda i: (0, 0)),
            pl.BlockSpec((1, 128), lambda i: (0, 0)),
            pl.BlockSpec((384, 128), lambda i: (0, 0)),
            pl.BlockSpec((384, 128), lambda i: (0, 0)),
            pl.BlockSpec((1, 128), lambda i: (0, 0)),
            pl.BlockSpec((3200, 128), lambda i: (0, 0)),
            pl.BlockSpec((1, 128), lambda i: (0, 0)),
            pl.BlockSpec((128, C), lambda i: (0, 0)),
            pl.BlockSpec((1, C), lambda i: (0, 0)),
        ],
        out_specs=pl.BlockSpec((bb, C), lambda i: (i, 0)),
        scratch_shapes=[
            pltpu.VMEM((ch, 100, 48), bf16),
            pltpu.VMEM((ch, 100, 128), jnp.float32),
            pltpu.VMEM((ch, 50, 384), bf16),
            pltpu.VMEM((ch, 50, 128), jnp.float32),
            pltpu.VMEM((bb, 3200), bf16),
        ],
        compiler_params=pltpu.CompilerParams(
            dimension_semantics=("parallel",)),
    )(xt, a1e, a1o, b1t, a2e, a2o, b2t, wf1r, bf1, wf2.astype(bf16), bf2)
    return out
